# baseline (speedup 1.0000x reference)
import jax
import jax.numpy as jnp
from jax import lax
from jax.experimental import pallas as pl
from jax.experimental.pallas import tpu as pltpu
from jax.experimental.pallas import tpu_sc as plsc

DIMS = (8, 16, 32, 64)
FOLD = (16, 8, 4, 2)
SHIFT = (4, 3, 2, 1)
V = 100001
VP = 100096
B = 16384
NC, NS = 2, 16
NW = NC * NS
BPW = B // NW
NCH = BPW // 128
ROW_BLK = 2048


def _sc_gather_body(x_hbm, w0, w1, w2, w3, e4,
                    og0, og1, og2, og3, o4,
                    idx_v, g0i, g1i, g2i, g3i,
                    ra0, ra1, ra2, ra3, ra4,
                    rb0, rb1, rb2, rb3, rb4, gsem, wsem):
    wid = lax.axis_index("s") * NC + lax.axis_index("c")
    base = wid * BPW
    pltpu.sync_copy(x_hbm.at[pl.ds(wid * NCH, NCH)], idx_v)
    for gi, sh in zip((g0i, g1i, g2i, g3i), SHIFT):
        for j in range(NCH):
            for k in range(8):
                v16 = idx_v[j, pl.ds(k * 16, 16)]
                gi[j, pl.ds(k * 16, 16)] = jnp.right_shift(v16, sh)
    tables = (w0, w1, w2, w3, e4)
    idxs = (g0i, g1i, g2i, g3i, idx_v)
    bufs = ((ra0, ra1, ra2, ra3, ra4), (rb0, rb1, rb2, rb3, rb4))
    outs = (og0, og1, og2, og3, o4)
    nchunk = 2 * NCH

    def fire_gathers(c):
        j, h = c // 2, (c % 2) * 64
        return [pltpu.async_copy(t.at[gi.at[j, pl.ds(h, 64)]], rb, gsem)
                for t, gi, rb in zip(tables, idxs, bufs[c % 2])]

    def fire_writes(c):
        j, h = c // 2, (c % 2) * 64
        return [pltpu.async_copy(rb, o.at[pl.ds(base + j * 128 + h, 64)],
                                 wsem)
                for rb, o in zip(bufs[c % 2], outs)]

    gath = fire_gathers(0)
    writes = {}
    for c in range(nchunk):
        if c + 1 < nchunk:
            if c - 1 >= 0:
                for cp in writes.pop(c - 1):
                    cp.wait()
            nxt = fire_gathers(c + 1)
        else:
            nxt = None
        for cp in gath:
            cp.wait()
        writes[c] = fire_writes(c)
        gath = nxt
    for c, ws in writes.items():
        for cp in ws:
            cp.wait()


def _sc_gather(x2d, w0, w1, w2, w3, e4):
    mesh = plsc.VectorSubcoreMesh(core_axis_name="c", subcore_axis_name="s")
    return pl.kernel(
        _sc_gather_body,
        out_type=tuple(jax.ShapeDtypeStruct((B, 128), jnp.float32)
                       for _ in range(5)),
        mesh=mesh,
        scratch_types=[pltpu.VMEM((NCH, 128), jnp.int32)] * 5
        + [pltpu.VMEM((64, 128), jnp.float32)] * 10
        + [pltpu.SemaphoreType.DMA] * 2,
    )(x2d, w0, w1, w2, w3, e4)


def _tc_combine_body(dw_ref, x_ref, g0, g1, g2, g3, o4,
                     r0, r1, r2, r3, bstack_ref, out_ref):
    w = [dw_ref[i] for i in range(5)]
    xcol = x_ref[...]
    col = lax.broadcasted_iota(jnp.int32, (ROW_BLK, 128), 1)
    acc = w[4] * o4[...]
    for ti, (g, rr, d, f) in enumerate(
            zip((g0, g1, g2, g3), (r0, r1, r2, r3), DIMS, FOLD)):
        lo = jnp.bitwise_and(xcol, f - 1) * d
        m = jnp.logical_and(col >= lo, col < lo + d).astype(jnp.float32)
        acc += w[ti] * jnp.dot(g[...] * m, rr[...],
                               preferred_element_type=jnp.float32)
    bias = (w[0] * bstack_ref[0, :] + w[1] * bstack_ref[1, :]
            + w[2] * bstack_ref[2, :] + w[3] * bstack_ref[3, :])
    out_ref[...] = acc + bias[None, :]


def _tc_combine(xcol, gs, reps, bstack, dw):
    return pl.pallas_call(
        _tc_combine_body,
        grid=(B // ROW_BLK,),
        in_specs=[pl.BlockSpec(memory_space=pltpu.SMEM),
                  pl.BlockSpec((ROW_BLK, 1), lambda i: (i, 0))]
        + [pl.BlockSpec((ROW_BLK, 128), lambda i: (i, 0))] * 5
        + [pl.BlockSpec((128, 128), lambda i: (0, 0))] * 4
        + [pl.BlockSpec((4, 128), lambda i: (0, 0))],
        out_specs=pl.BlockSpec((ROW_BLK, 128), lambda i: (i, 0)),
        out_shape=jax.ShapeDtypeStruct((B, 128), jnp.float32),
    )(dw, xcol, *gs, *reps, bstack)


def kernel(x, emb_0, emb_1, emb_2, emb_3, emb_4,
           proj_w_0, proj_b_0, proj_w_1, proj_b_1,
           proj_w_2, proj_b_2, proj_w_3, proj_b_3,
           dim_logits):
    dim_weights = jax.nn.softmax(dim_logits, axis=-1)
    folded = []
    for t, d in zip((emb_0, emb_1, emb_2, emb_3, emb_4), DIMS + (128,)):
        flat = jnp.concatenate(
            [t.reshape(-1), jnp.zeros(((VP - V) * d,), jnp.float32)])
        folded.append(flat.reshape(VP * d // 128, 128))
    xi = x.astype(jnp.int32)
    x2d = xi.reshape(B // 128, 128)
    gs = _sc_gather(x2d, *folded)
    bstack = jnp.stack([proj_b_0, proj_b_1, proj_b_2, proj_b_3], axis=0)
    reps = tuple(jnp.tile(pw.T, (f, 1))
                 for pw, f in zip((proj_w_0, proj_w_1, proj_w_2, proj_w_3),
                                  FOLD))
    out = _tc_combine(xi.reshape(B, 1), gs, reps, bstack, dim_weights)
    return (out, dim_weights)


# speedup vs baseline: 1.4421x; 1.4421x over previous
import jax
import jax.numpy as jnp
from jax import lax
from jax.experimental import pallas as pl
from jax.experimental.pallas import tpu as pltpu
from jax.experimental.pallas import tpu_sc as plsc

DIMS = (8, 16, 32, 64, 128)
OFFS = (0, 8, 24, 56)
B = 16384
NC, NS = 2, 16
NW = NC * NS
BPW = B // NW
NCH = BPW // 128
ROW_BLK = 2048


def _sc_gather_body(x_hbm, e0, e1, e2, e3, e4,
                    ocat, o4,
                    idx_v, r0, r1, r2, r3, r4, sem):
    wid = lax.axis_index("s") * NC + lax.axis_index("c")
    base = wid * BPW
    pltpu.sync_copy(x_hbm.at[pl.ds(wid * NCH, NCH)], idx_v)
    bufs = (r0, r1, r2, r3, r4)
    copies = []
    for t, rb in zip((e0, e1, e2, e3, e4), bufs):
        for j in range(NCH):
            copies.append(pltpu.async_copy(
                t.at[idx_v.at[j]], rb.at[pl.ds(j * 128, 128)], sem))
    for c in copies:
        c.wait()
    outs = []
    for rb, off, d in zip(bufs, OFFS, DIMS):
        outs.append(pltpu.async_copy(
            rb, ocat.at[pl.ds(base, BPW), pl.ds(off, d)], sem))
    outs.append(pltpu.async_copy(r4, o4.at[pl.ds(base, BPW)], sem))
    for c in outs:
        c.wait()


def _sc_gather(x2d, e0, e1, e2, e3, e4):
    mesh = plsc.VectorSubcoreMesh(core_axis_name="c", subcore_axis_name="s")
    return pl.kernel(
        _sc_gather_body,
        out_type=(jax.ShapeDtypeStruct((B, 128), jnp.float32),
                  jax.ShapeDtypeStruct((B, 128), jnp.float32)),
        mesh=mesh,
        scratch_types=[
            pltpu.VMEM((NCH, 128), jnp.int32),
            pltpu.VMEM((BPW, 8), jnp.float32),
            pltpu.VMEM((BPW, 16), jnp.float32),
            pltpu.VMEM((BPW, 32), jnp.float32),
            pltpu.VMEM((BPW, 64), jnp.float32),
            pltpu.VMEM((BPW, 128), jnp.float32),
            pltpu.SemaphoreType.DMA,
        ],
        compiler_params=pltpu.CompilerParams(use_tc_tiling_on_sc=False),
    )(x2d, e0, e1, e2, e3, e4)


def _tc_combine_body(dw_ref, cat_ref, e4_ref, pcat_ref, bstack_ref, out_ref):
    w = [dw_ref[i] for i in range(5)]
    cat = cat_ref[...]
    col = lax.broadcasted_iota(jnp.int32, (ROW_BLK, 128), 1)
    cat = jnp.where(col < 120, cat, 0.0)
    r = lax.broadcasted_iota(jnp.int32, (128, 128), 0)
    scale = jnp.where(r < 8, w[0],
            jnp.where(r < 24, w[1],
            jnp.where(r < 56, w[2],
            jnp.where(r < 120, w[3], 0.0))))
    p = pcat_ref[...] * scale
    acc = jnp.dot(cat, p, preferred_element_type=jnp.float32)
    bias = (w[0] * bstack_ref[0, :] + w[1] * bstack_ref[1, :]
            + w[2] * bstack_ref[2, :] + w[3] * bstack_ref[3, :])
    out_ref[...] = acc + bias[None, :] + w[4] * e4_ref[...]


def _tc_combine(cat, e4, pcat, bstack, dw):
    return pl.pallas_call(
        _tc_combine_body,
        grid=(B // ROW_BLK,),
        in_specs=[
            pl.BlockSpec(memory_space=pltpu.SMEM),
            pl.BlockSpec((ROW_BLK, 128), lambda i: (i, 0)),
            pl.BlockSpec((ROW_BLK, 128), lambda i: (i, 0)),
            pl.BlockSpec((128, 128), lambda i: (0, 0)),
            pl.BlockSpec((4, 128), lambda i: (0, 0)),
        ],
        out_specs=pl.BlockSpec((ROW_BLK, 128), lambda i: (i, 0)),
        out_shape=jax.ShapeDtypeStruct((B, 128), jnp.float32),
    )(dw, cat, e4, pcat, bstack)


def kernel(x, emb_0, emb_1, emb_2, emb_3, emb_4,
           proj_w_0, proj_b_0, proj_w_1, proj_b_1,
           proj_w_2, proj_b_2, proj_w_3, proj_b_3,
           dim_logits):
    dim_weights = jax.nn.softmax(dim_logits, axis=-1)
    x2d = x.astype(jnp.int32).reshape(B // 128, 128)
    cat, e4 = _sc_gather(x2d, emb_0, emb_1, emb_2, emb_3, emb_4)
    pcat = jnp.concatenate(
        [proj_w_0.T, proj_w_1.T, proj_w_2.T, proj_w_3.T,
         jnp.zeros((8, 128), jnp.float32)], axis=0)
    bstack = jnp.stack([proj_b_0, proj_b_1, proj_b_2, proj_b_3], axis=0)
    out = _tc_combine(cat, e4, pcat, bstack, dim_weights)
    return (out, dim_weights)


# speedup vs baseline: 1.4533x; 1.0078x over previous
import jax
import jax.numpy as jnp
from jax import lax
from jax.experimental import pallas as pl
from jax.experimental.pallas import tpu as pltpu
from jax.experimental.pallas import tpu_sc as plsc

DIMS = (8, 16, 32, 64, 128)
OFFS = (0, 8, 24, 56)
B = 16384
NC, NS = 2, 16
NW = NC * NS
BPW = B // NW
NCH = BPW // 128
ROW_BLK = 2048


def _sc_gather_body(x_hbm, e0, e1, e2, e3,
                    ocat,
                    idx_v, r0, r1, r2, r3, sem):
    wid = lax.axis_index("s") * NC + lax.axis_index("c")
    base = wid * BPW
    pltpu.sync_copy(x_hbm.at[pl.ds(wid * NCH, NCH)], idx_v)
    bufs = (r0, r1, r2, r3)
    copies = []
    for t, rb in zip((e0, e1, e2, e3), bufs):
        for j in range(NCH):
            copies.append(pltpu.async_copy(
                t.at[idx_v.at[j]], rb.at[pl.ds(j * 128, 128)], sem))
    for c in copies:
        c.wait()
    outs = []
    for rb, off, d in zip(bufs, OFFS, DIMS):
        outs.append(pltpu.async_copy(
            rb, ocat.at[pl.ds(base, BPW), pl.ds(off, d)], sem))
    for c in outs:
        c.wait()


def _sc_gather(x2d, e0, e1, e2, e3):
    mesh = plsc.VectorSubcoreMesh(core_axis_name="c", subcore_axis_name="s")
    return pl.kernel(
        _sc_gather_body,
        out_type=jax.ShapeDtypeStruct((B, 128), jnp.float32),
        mesh=mesh,
        scratch_types=[
            pltpu.VMEM((NCH, 128), jnp.int32),
            pltpu.VMEM((BPW, 8), jnp.float32),
            pltpu.VMEM((BPW, 16), jnp.float32),
            pltpu.VMEM((BPW, 32), jnp.float32),
            pltpu.VMEM((BPW, 64), jnp.float32),
            pltpu.SemaphoreType.DMA,
        ],
        compiler_params=pltpu.CompilerParams(use_tc_tiling_on_sc=False),
    )(x2d, e0, e1, e2, e3)


def _sc_gather_e4_body(x_hbm, e4, o4, idx_v, ra, rb, rc, rd, sem):
    wid = lax.axis_index("s") * NC + lax.axis_index("c")
    base = wid * BPW
    pltpu.sync_copy(x_hbm.at[pl.ds(wid * NCH, NCH)], idx_v)
    bufs = (ra, rb, rc, rd)
    copies = [pltpu.async_copy(e4.at[idx_v.at[j]], bufs[j], sem)
              for j in range(NCH)]
    for c in copies:
        c.wait()
    outs = [pltpu.async_copy(bufs[j], o4.at[pl.ds(base + j * 128, 128)], sem)
            for j in range(NCH)]
    for c in outs:
        c.wait()


def _sc_gather_e4(x2d, e4):
    mesh = plsc.VectorSubcoreMesh(core_axis_name="c", subcore_axis_name="s")
    return pl.kernel(
        _sc_gather_e4_body,
        out_type=jax.ShapeDtypeStruct((B, 128), jnp.float32),
        mesh=mesh,
        scratch_types=[pltpu.VMEM((NCH, 128), jnp.int32)]
        + [pltpu.VMEM((128, 128), jnp.float32)] * 4
        + [pltpu.SemaphoreType.DMA],
    )(x2d, e4)


def _tc_combine_body(dw_ref, cat_ref, e4_ref, pcat_ref, bstack_ref, out_ref):
    w = [dw_ref[i] for i in range(5)]
    cat = cat_ref[...]
    col = lax.broadcasted_iota(jnp.int32, (ROW_BLK, 128), 1)
    cat = jnp.where(col < 120, cat, 0.0)
    r = lax.broadcasted_iota(jnp.int32, (128, 128), 0)
    scale = jnp.where(r < 8, w[0],
            jnp.where(r < 24, w[1],
            jnp.where(r < 56, w[2],
            jnp.where(r < 120, w[3], 0.0))))
    p = pcat_ref[...] * scale
    acc = jnp.dot(cat, p, preferred_element_type=jnp.float32)
    bias = (w[0] * bstack_ref[0, :] + w[1] * bstack_ref[1, :]
            + w[2] * bstack_ref[2, :] + w[3] * bstack_ref[3, :])
    out_ref[...] = acc + bias[None, :] + w[4] * e4_ref[...]


def _tc_combine(cat, e4, pcat, bstack, dw):
    return pl.pallas_call(
        _tc_combine_body,
        grid=(B // ROW_BLK,),
        in_specs=[
            pl.BlockSpec(memory_space=pltpu.SMEM),
            pl.BlockSpec((ROW_BLK, 128), lambda i: (i, 0)),
            pl.BlockSpec((ROW_BLK, 128), lambda i: (i, 0)),
            pl.BlockSpec((128, 128), lambda i: (0, 0)),
            pl.BlockSpec((4, 128), lambda i: (0, 0)),
        ],
        out_specs=pl.BlockSpec((ROW_BLK, 128), lambda i: (i, 0)),
        out_shape=jax.ShapeDtypeStruct((B, 128), jnp.float32),
    )(dw, cat, e4, pcat, bstack)


def kernel(x, emb_0, emb_1, emb_2, emb_3, emb_4,
           proj_w_0, proj_b_0, proj_w_1, proj_b_1,
           proj_w_2, proj_b_2, proj_w_3, proj_b_3,
           dim_logits):
    dim_weights = jax.nn.softmax(dim_logits, axis=-1)
    x2d = x.astype(jnp.int32).reshape(B // 128, 128)
    cat = _sc_gather(x2d, emb_0, emb_1, emb_2, emb_3)
    e4 = _sc_gather_e4(x2d, emb_4)
    pcat = jnp.concatenate(
        [proj_w_0.T, proj_w_1.T, proj_w_2.T, proj_w_3.T,
         jnp.zeros((8, 128), jnp.float32)], axis=0)
    bstack = jnp.stack([proj_b_0, proj_b_1, proj_b_2, proj_b_3], axis=0)
    out = _tc_combine(cat, e4, pcat, bstack, dim_weights)
    return (out, dim_weights)
